# baseline (speedup 1.0000x reference)
import functools

import jax
import jax.numpy as jnp
from jax.experimental import pallas as pl
from jax.experimental.pallas import tpu as pltpu
from jax.experimental.pallas import tpu_sc as plsc

HIDDEN = 1024
NUM_EXPERTS = 8
TOP_K = 2
MOE_FF = 512

BT = 256
BLK = 256
T_TOK = 2048
A = T_TOK * TOP_K
NB_MAX = A // BLK + NUM_EXPERTS
NP_MAX = NB_MAX * BLK

_NC, _NS = 2, 16
_NW = _NC * _NS
_CH = 64


def _pack_halves16(a16):
    n = a16.shape[1] // 2
    lo = jax.lax.bitcast_convert_type(a16[:, :n], jnp.uint16).astype(jnp.uint32)
    hi = jax.lax.bitcast_convert_type(a16[:, n:], jnp.uint16).astype(jnp.uint32)
    return jax.lax.bitcast_convert_type((hi << 16) | lo, jnp.int32)


def _router_kernel(x_ref, gw_ref, w_ref, pos_ref, counts_ref, xpk_ref, run_ref):
    t = pl.program_id(0)

    @pl.when(t == 0)
    def _():
        run_ref[...] = jnp.zeros((1, NUM_EXPERTS), jnp.float32)

    xb = x_ref[...]
    logits = jnp.dot(xb, gw_ref[...].T, preferred_element_type=jnp.float32)
    m = jnp.max(logits, axis=1, keepdims=True)
    p = jnp.exp(logits - m)
    prob = p / jnp.sum(p, axis=1, keepdims=True)
    iota_e = jax.lax.broadcasted_iota(jnp.int32, prob.shape, 1)
    i1 = jnp.argmax(prob, axis=1).astype(jnp.int32)
    w1 = jnp.max(prob, axis=1)
    masked = jnp.where(iota_e == i1[:, None], -1.0, prob)
    i2 = jnp.argmax(masked, axis=1).astype(jnp.int32)
    w2 = jnp.max(masked, axis=1)
    s = w1 + w2
    w_ref[...] = jnp.stack([w1 / s, w2 / s], axis=1)
    xpk_ref[...] = _pack_halves16(xb.astype(jnp.bfloat16))

    oh1 = (iota_e == i1[:, None]).astype(jnp.float32)
    oh2 = (iota_e == i2[:, None]).astype(jnp.float32)
    cnt = oh1 + oh2
    r_i = jax.lax.broadcasted_iota(jnp.int32, (BT, BT), 0)
    c_i = jax.lax.broadcasted_iota(jnp.int32, (BT, BT), 1)
    tril = (c_i < r_i).astype(jnp.float32)
    excl = jnp.dot(tril, cnt, preferred_element_type=jnp.float32) + run_ref[...]
    rank1 = jnp.sum(oh1 * excl, axis=1)
    rank2 = jnp.sum(oh2 * excl, axis=1)
    pos1 = i1 * T_TOK + rank1.astype(jnp.int32)
    pos2 = i2 * T_TOK + rank2.astype(jnp.int32)
    pos_ref[...] = jnp.stack([pos1, pos2], axis=1)
    run_new = run_ref[...] + jnp.sum(cnt, axis=0, keepdims=True)
    run_ref[...] = run_new
    counts_ref[...] = run_new.astype(jnp.int32)


def _router(x, gate_w):
    T, H = x.shape
    E = NUM_EXPERTS
    return pl.pallas_call(
        _router_kernel,
        grid=(T // BT,),
        in_specs=[
            pl.BlockSpec((BT, H), lambda t: (t, 0)),
            pl.BlockSpec((E, H), lambda t: (0, 0)),
        ],
        out_specs=[
            pl.BlockSpec((BT, TOP_K), lambda t: (t, 0)),
            pl.BlockSpec((BT, TOP_K), lambda t: (t, 0)),
            pl.BlockSpec((1, E), lambda t: (0, 0)),
            pl.BlockSpec((BT, H // 2), lambda t: (t, 0)),
        ],
        out_shape=[
            jax.ShapeDtypeStruct((T, TOP_K), jnp.float32),
            jax.ShapeDtypeStruct((T, TOP_K), jnp.int32),
            jax.ShapeDtypeStruct((1, E), jnp.int32),
            jax.ShapeDtypeStruct((T, H // 2), jnp.int32),
        ],
        scratch_shapes=[pltpu.VMEM((1, E), jnp.float32)],
    )(x, gate_w)


def _plan(counts):
    c = counts[0]
    nact = (c + BLK - 1) // BLK
    cend = jnp.cumsum(nact)
    cstart = cend - nact
    nblocks = cend[-1].astype(jnp.int32)
    s = jnp.arange(NB_MAX, dtype=jnp.int32)
    sc = jnp.minimum(s, nblocks - 1)
    block_e = jnp.sum(sc[:, None] >= cend[None, :], axis=1).astype(jnp.int32)
    bidx = (block_e * (T_TOK // BLK) + sc - cstart[block_e]).astype(jnp.int32)
    return nblocks, block_e, bidx


def _sc_dispatch(x, pos0, pos1):
    T, H = x.shape
    bpw = T // _NW
    mesh = plsc.VectorSubcoreMesh(core_axis_name="c", subcore_axis_name="s")

    @functools.partial(
        pl.kernel, mesh=mesh,
        out_type=jax.ShapeDtypeStruct((NUM_EXPERTS * T_TOK, H), x.dtype),

        scratch_types=[
            pltpu.VMEM((bpw,), jnp.int32),
            pltpu.VMEM((bpw,), jnp.int32),
            pltpu.VMEM((bpw, H), x.dtype),
            pltpu.SemaphoreType.DMA,
        ],
    )
    def k(x_hbm, p0_hbm, p1_hbm, xs_hbm, i0_v, i1_v, rows_v, sem):
        wid = jax.lax.axis_index("s") * _NC + jax.lax.axis_index("c")
        base = wid * bpw
        pltpu.sync_copy(p0_hbm.at[pl.ds(base, bpw)], i0_v)
        pltpu.sync_copy(p1_hbm.at[pl.ds(base, bpw)], i1_v)
        pltpu.sync_copy(x_hbm.at[pl.ds(base, bpw)], rows_v)
        pltpu.async_copy(rows_v, xs_hbm.at[i0_v], sem).wait()
        pltpu.async_copy(rows_v, xs_hbm.at[i1_v], sem).wait()

    return k(x, pos0, pos1)


def _sc_gather(table, idx):
    B = idx.shape[0]
    V, D = table.shape
    bpw = B // _NW
    nch = bpw // _CH
    mesh = plsc.VectorSubcoreMesh(core_axis_name="c", subcore_axis_name="s")

    @functools.partial(
        pl.kernel, mesh=mesh,
        out_type=jax.ShapeDtypeStruct((B, D), table.dtype),
        scratch_types=[
            pltpu.VMEM((_CH,), jnp.int32),
            pltpu.VMEM((_CH, D), table.dtype),
            pltpu.SemaphoreType.DMA,
        ],
    )
    def k(table_hbm, idx_hbm, out_hbm, idx_v, rows_v, sem):
        wid = jax.lax.axis_index("s") * _NC + jax.lax.axis_index("c")
        base = wid * bpw
        for c in range(nch):
            off = base + c * _CH
            pltpu.sync_copy(idx_hbm.at[pl.ds(off, _CH)], idx_v)
            pltpu.async_copy(table_hbm.at[idx_v], rows_v, sem).wait()
            pltpu.sync_copy(rows_v, out_hbm.at[pl.ds(off, _CH)])

    return k(table, idx)


def _gm_kernel(be_ref, bi_ref, nb_ref, xs_ref, wg_ref, wu_ref, wd_ref, y_ref,
               acc_ref):
    i = pl.program_id(0)
    h = pl.program_id(1)

    @pl.when(i < nb_ref[0])
    def _():
        v = jax.lax.bitcast_convert_type(xs_ref[...], jnp.uint32)
        xlo = jax.lax.bitcast_convert_type(
            (v & 0xFFFF).astype(jnp.uint16), jnp.bfloat16)
        xhi = jax.lax.bitcast_convert_type(
            (v >> 16).astype(jnp.uint16), jnp.bfloat16)
        xb = jnp.concatenate([xlo, xhi], axis=1)
        g = jnp.dot(xb, wg_ref[0].astype(jnp.bfloat16).T,
                    preferred_element_type=jnp.float32)
        u = jnp.dot(xb, wu_ref[0].astype(jnp.bfloat16).T,
                    preferred_element_type=jnp.float32)
        act = g * jax.nn.sigmoid(g) * u
        o = jnp.dot(act.astype(jnp.bfloat16), wd_ref[0].astype(jnp.bfloat16).T,
                    preferred_element_type=jnp.float32)

        @pl.when(h == 0)
        def _():
            acc_ref[...] = o

        @pl.when(h == 1)
        def _():
            o16 = (acc_ref[...] + o).astype(jnp.bfloat16)
            lo = jax.lax.bitcast_convert_type(
                o16[:, :HIDDEN // 2], jnp.uint16).astype(jnp.uint32)
            hi = jax.lax.bitcast_convert_type(
                o16[:, HIDDEN // 2:], jnp.uint16).astype(jnp.uint32)
            y_ref[...] = jax.lax.bitcast_convert_type((hi << 16) | lo,
                                                      jnp.int32)


def _grouped_mlp(xs, Wg, Wu, Wd, nblocks, block_e, bidx):
    H, F = HIDDEN, MOE_FF
    grid_spec = pltpu.PrefetchScalarGridSpec(
        num_scalar_prefetch=3,
        grid=(NB_MAX, 2),
        in_specs=[
            pl.BlockSpec((BLK, H // 2), lambda i, h, be, bi, nb: (bi[i], 0)),
            pl.BlockSpec((1, F // 2, H), lambda i, h, be, bi, nb: (be[i], h, 0)),
            pl.BlockSpec((1, F // 2, H), lambda i, h, be, bi, nb: (be[i], h, 0)),
            pl.BlockSpec((1, H, F // 2), lambda i, h, be, bi, nb: (be[i], 0, h)),
        ],
        out_specs=pl.BlockSpec((BLK, H // 2), lambda i, h, be, bi, nb: (bi[i], 0)),
        scratch_shapes=[pltpu.VMEM((BLK, H), jnp.float32)],
    )
    return pl.pallas_call(
        _gm_kernel,
        grid_spec=grid_spec,
        out_shape=jax.ShapeDtypeStruct((NUM_EXPERTS * T_TOK, H // 2), jnp.int32),
    )(block_e, bidx, nblocks.reshape(1), xs, Wg, Wu, Wd)


def _unpack_halves(v_i32):
    v = jax.lax.bitcast_convert_type(v_i32, jnp.uint32)
    lo = jax.lax.bitcast_convert_type(
        (v & 0xFFFF).astype(jnp.uint16), jnp.bfloat16).astype(jnp.float32)
    hi = jax.lax.bitcast_convert_type(
        (v >> 16).astype(jnp.uint16), jnp.bfloat16).astype(jnp.float32)
    return lo, hi


def _combine_kernel(w_ref, y0_ref, y1_ref, out_ref):
    lo0, hi0 = _unpack_halves(y0_ref[...])
    lo1, hi1 = _unpack_halves(y1_ref[...])
    w0 = w_ref[:, 0:1]
    w1 = w_ref[:, 1:2]
    out_ref[:, :HIDDEN // 2] = w0 * lo0 + w1 * lo1
    out_ref[:, HIDDEN // 2:] = w0 * hi0 + w1 * hi1


def _combine(w, yy, T):
    Hw = yy.shape[1]
    nt = T // BT
    return pl.pallas_call(
        _combine_kernel,
        grid=(nt,),
        in_specs=[
            pl.BlockSpec((BT, TOP_K), lambda t: (t, 0)),
            pl.BlockSpec((BT, Hw), lambda t: (t, 0)),
            pl.BlockSpec((BT, Hw), lambda t, _nt=nt: (t + _nt, 0)),
        ],
        out_specs=pl.BlockSpec((BT, HIDDEN), lambda t: (t, 0)),
        out_shape=jax.ShapeDtypeStruct((T, HIDDEN), jnp.float32),
    )(w, yy, yy)


@jax.jit
def _moe(x, gate_w, Wg, Wu, Wd):
    T = x.shape[0]
    w, pos, counts, xpk = _router(x, gate_w)
    nblocks, block_e, bidx = _plan(counts)
    xs = _sc_dispatch(xpk, pos[:, 0], pos[:, 1])
    y = _grouped_mlp(xs, Wg, Wu, Wd, nblocks, block_e, bidx)
    yy = _sc_gather(y, pos.T.reshape(-1))
    return _combine(w, yy, T)


def kernel(hidden_states, gate_w, Wg, Wu, Wd):
    b, s, h = hidden_states.shape
    x = hidden_states.reshape(-1, h)
    out = _moe(x, gate_w, Wg, Wu, Wd)
    return out.reshape(b, s, h)


# speedup vs baseline: 1.2825x; 1.2825x over previous
import functools

import jax
import jax.numpy as jnp
from jax.experimental import pallas as pl
from jax.experimental.pallas import tpu as pltpu
from jax.experimental.pallas import tpu_sc as plsc

HIDDEN = 1024
NUM_EXPERTS = 8
TOP_K = 2
MOE_FF = 512

BT = 256
BLK = 256
T_TOK = 2048
A = T_TOK * TOP_K
NB_MAX = A // BLK + NUM_EXPERTS
NP_MAX = NB_MAX * BLK

_NC, _NS = 2, 16
_NW = _NC * _NS
_CH = 64


def _pack_halves16(a16):
    n = a16.shape[1] // 2
    lo = jax.lax.bitcast_convert_type(a16[:, :n], jnp.uint16).astype(jnp.uint32)
    hi = jax.lax.bitcast_convert_type(a16[:, n:], jnp.uint16).astype(jnp.uint32)
    return jax.lax.bitcast_convert_type((hi << 16) | lo, jnp.int32)


def _router_kernel(x_ref, gw_ref, w_ref, pos_ref, counts_ref, xpk_ref, run_ref):
    t = pl.program_id(0)

    @pl.when(t == 0)
    def _():
        run_ref[...] = jnp.zeros((1, NUM_EXPERTS), jnp.float32)

    xb = x_ref[...]
    logits = jnp.dot(xb, gw_ref[...].T, preferred_element_type=jnp.float32)
    m = jnp.max(logits, axis=1, keepdims=True)
    p = jnp.exp(logits - m)
    prob = p / jnp.sum(p, axis=1, keepdims=True)
    iota_e = jax.lax.broadcasted_iota(jnp.int32, prob.shape, 1)
    i1 = jnp.argmax(prob, axis=1).astype(jnp.int32)
    w1 = jnp.max(prob, axis=1)
    masked = jnp.where(iota_e == i1[:, None], -1.0, prob)
    i2 = jnp.argmax(masked, axis=1).astype(jnp.int32)
    w2 = jnp.max(masked, axis=1)
    s = w1 + w2
    w_ref[...] = jnp.stack([w1 / s, w2 / s], axis=1)
    xpk_ref[...] = _pack_halves16(xb.astype(jnp.bfloat16))

    oh1 = (iota_e == i1[:, None]).astype(jnp.float32)
    oh2 = (iota_e == i2[:, None]).astype(jnp.float32)
    cnt = oh1 + oh2
    r_i = jax.lax.broadcasted_iota(jnp.int32, (BT, BT), 0)
    c_i = jax.lax.broadcasted_iota(jnp.int32, (BT, BT), 1)
    tril = (c_i < r_i).astype(jnp.float32)
    excl = jnp.dot(tril, cnt, preferred_element_type=jnp.float32) + run_ref[...]
    rank1 = jnp.sum(oh1 * excl, axis=1)
    rank2 = jnp.sum(oh2 * excl, axis=1)
    pos1 = i1 * T_TOK + rank1.astype(jnp.int32)
    pos2 = i2 * T_TOK + rank2.astype(jnp.int32)
    pos_ref[...] = jnp.stack([pos1, pos2], axis=1)
    run_new = run_ref[...] + jnp.sum(cnt, axis=0, keepdims=True)
    run_ref[...] = run_new
    counts_ref[...] = run_new.astype(jnp.int32)


def _router(x, gate_w):
    T, H = x.shape
    E = NUM_EXPERTS
    return pl.pallas_call(
        _router_kernel,
        grid=(T // BT,),
        in_specs=[
            pl.BlockSpec((BT, H), lambda t: (t, 0)),
            pl.BlockSpec((E, H), lambda t: (0, 0)),
        ],
        out_specs=[
            pl.BlockSpec((BT, TOP_K), lambda t: (t, 0)),
            pl.BlockSpec((BT, TOP_K), lambda t: (t, 0)),
            pl.BlockSpec((1, E), lambda t: (0, 0)),
            pl.BlockSpec((BT, H // 2), lambda t: (t, 0)),
        ],
        out_shape=[
            jax.ShapeDtypeStruct((T, TOP_K), jnp.float32),
            jax.ShapeDtypeStruct((T, TOP_K), jnp.int32),
            jax.ShapeDtypeStruct((1, E), jnp.int32),
            jax.ShapeDtypeStruct((T, H // 2), jnp.int32),
        ],
        scratch_shapes=[pltpu.VMEM((1, E), jnp.float32)],
    )(x, gate_w)


def _plan(counts):
    c = counts[0]
    nact = (c + BLK - 1) // BLK
    cend = jnp.cumsum(nact)
    cstart = cend - nact
    nblocks = cend[-1].astype(jnp.int32)
    s = jnp.arange(NB_MAX, dtype=jnp.int32)
    sc = jnp.minimum(s, nblocks - 1)
    block_e = jnp.sum(sc[:, None] >= cend[None, :], axis=1).astype(jnp.int32)
    bidx = (block_e * (T_TOK // BLK) + sc - cstart[block_e]).astype(jnp.int32)
    return nblocks, block_e, bidx


def _sc_dispatch(x, pos0, pos1):
    T, H = x.shape
    bpw = T // _NW
    mesh = plsc.VectorSubcoreMesh(core_axis_name="c", subcore_axis_name="s")

    @functools.partial(
        pl.kernel, mesh=mesh,
        out_type=jax.ShapeDtypeStruct((NUM_EXPERTS * T_TOK, H), x.dtype),

        scratch_types=[
            pltpu.VMEM((bpw,), jnp.int32),
            pltpu.VMEM((bpw,), jnp.int32),
            pltpu.VMEM((bpw, H), x.dtype),
            pltpu.SemaphoreType.DMA,
        ],
    )
    def k(x_hbm, p0_hbm, p1_hbm, xs_hbm, i0_v, i1_v, rows_v, sem):
        wid = jax.lax.axis_index("s") * _NC + jax.lax.axis_index("c")
        base = wid * bpw
        pltpu.sync_copy(p0_hbm.at[pl.ds(base, bpw)], i0_v)
        pltpu.sync_copy(p1_hbm.at[pl.ds(base, bpw)], i1_v)
        pltpu.sync_copy(x_hbm.at[pl.ds(base, bpw)], rows_v)
        pltpu.async_copy(rows_v, xs_hbm.at[i0_v], sem).wait()
        pltpu.async_copy(rows_v, xs_hbm.at[i1_v], sem).wait()

    return k(x, pos0, pos1)


def _sc_gather(table, idx):
    B = idx.shape[0]
    V, D = table.shape
    bpw = B // _NW
    nch = bpw // _CH
    mesh = plsc.VectorSubcoreMesh(core_axis_name="c", subcore_axis_name="s")

    @functools.partial(
        pl.kernel, mesh=mesh,
        out_type=jax.ShapeDtypeStruct((B, D), table.dtype),
        scratch_types=[
            pltpu.VMEM((_CH,), jnp.int32),
            pltpu.VMEM((_CH, D), table.dtype),
            pltpu.SemaphoreType.DMA,
        ],
    )
    def k(table_hbm, idx_hbm, out_hbm, idx_v, rows_v, sem):
        wid = jax.lax.axis_index("s") * _NC + jax.lax.axis_index("c")
        base = wid * bpw
        for c in range(nch):
            off = base + c * _CH
            pltpu.sync_copy(idx_hbm.at[pl.ds(off, _CH)], idx_v)
            pltpu.async_copy(table_hbm.at[idx_v], rows_v, sem).wait()
            pltpu.sync_copy(rows_v, out_hbm.at[pl.ds(off, _CH)])

    return k(table, idx)


def _gm_kernel(be_ref, bi_ref, nb_ref, xs_ref, wg_ref, wu_ref, wd_ref, y_ref):
    i = pl.program_id(0)

    @pl.when(i < nb_ref[0])
    def _():
        v = jax.lax.bitcast_convert_type(xs_ref[...], jnp.uint32)
        xlo = jax.lax.bitcast_convert_type(
            (v & 0xFFFF).astype(jnp.uint16), jnp.bfloat16)
        xhi = jax.lax.bitcast_convert_type(
            (v >> 16).astype(jnp.uint16), jnp.bfloat16)
        xb = jnp.concatenate([xlo, xhi], axis=1)
        g = jnp.dot(xb, wg_ref[0].astype(jnp.bfloat16).T,
                    preferred_element_type=jnp.float32)
        u = jnp.dot(xb, wu_ref[0].astype(jnp.bfloat16).T,
                    preferred_element_type=jnp.float32)
        act = g * jax.nn.sigmoid(g) * u
        o = jnp.dot(act.astype(jnp.bfloat16), wd_ref[0].astype(jnp.bfloat16).T,
                    preferred_element_type=jnp.float32)
        o16 = o.astype(jnp.bfloat16)
        lo = jax.lax.bitcast_convert_type(
            o16[:, :HIDDEN // 2], jnp.uint16).astype(jnp.uint32)
        hi = jax.lax.bitcast_convert_type(
            o16[:, HIDDEN // 2:], jnp.uint16).astype(jnp.uint32)
        y_ref[...] = jax.lax.bitcast_convert_type((hi << 16) | lo, jnp.int32)


def _grouped_mlp(xs, Wg, Wu, Wd, nblocks, block_e, bidx):
    H, F = HIDDEN, MOE_FF
    grid_spec = pltpu.PrefetchScalarGridSpec(
        num_scalar_prefetch=3,
        grid=(NB_MAX,),
        in_specs=[
            pl.BlockSpec((BLK, H // 2), lambda i, be, bi, nb: (bi[i], 0)),
            pl.BlockSpec((1, F, H), lambda i, be, bi, nb: (be[i], 0, 0)),
            pl.BlockSpec((1, F, H), lambda i, be, bi, nb: (be[i], 0, 0)),
            pl.BlockSpec((1, H, F), lambda i, be, bi, nb: (be[i], 0, 0)),
        ],
        out_specs=pl.BlockSpec((BLK, H // 2), lambda i, be, bi, nb: (bi[i], 0)),
    )
    return pl.pallas_call(
        _gm_kernel,
        grid_spec=grid_spec,
        out_shape=jax.ShapeDtypeStruct((NUM_EXPERTS * T_TOK, H // 2), jnp.int32),
    )(block_e, bidx, nblocks.reshape(1), xs, Wg, Wu, Wd)


def _unpack_halves(v_i32):
    v = jax.lax.bitcast_convert_type(v_i32, jnp.uint32)
    lo = jax.lax.bitcast_convert_type(
        (v & 0xFFFF).astype(jnp.uint16), jnp.bfloat16).astype(jnp.float32)
    hi = jax.lax.bitcast_convert_type(
        (v >> 16).astype(jnp.uint16), jnp.bfloat16).astype(jnp.float32)
    return lo, hi


def _combine_kernel(w_ref, y0_ref, y1_ref, out_ref):
    lo0, hi0 = _unpack_halves(y0_ref[...])
    lo1, hi1 = _unpack_halves(y1_ref[...])
    w0 = w_ref[:, 0:1]
    w1 = w_ref[:, 1:2]
    out_ref[:, :HIDDEN // 2] = w0 * lo0 + w1 * lo1
    out_ref[:, HIDDEN // 2:] = w0 * hi0 + w1 * hi1


def _combine(w, yy, T):
    Hw = yy.shape[1]
    nt = T // BT
    return pl.pallas_call(
        _combine_kernel,
        grid=(nt,),
        in_specs=[
            pl.BlockSpec((BT, TOP_K), lambda t: (t, 0)),
            pl.BlockSpec((BT, Hw), lambda t: (t, 0)),
            pl.BlockSpec((BT, Hw), lambda t, _nt=nt: (t + _nt, 0)),
        ],
        out_specs=pl.BlockSpec((BT, HIDDEN), lambda t: (t, 0)),
        out_shape=jax.ShapeDtypeStruct((T, HIDDEN), jnp.float32),
    )(w, yy, yy)


@jax.jit
def _moe(x, gate_w, Wg, Wu, Wd):
    T = x.shape[0]
    w, pos, counts, xpk = _router(x, gate_w)
    nblocks, block_e, bidx = _plan(counts)
    xs = _sc_dispatch(xpk, pos[:, 0], pos[:, 1])
    y = _grouped_mlp(xs, Wg, Wu, Wd, nblocks, block_e, bidx)
    yy = _sc_gather(y, pos.T.reshape(-1))
    return _combine(w, yy, T)


def kernel(hidden_states, gate_w, Wg, Wu, Wd):
    b, s, h = hidden_states.shape
    x = hidden_states.reshape(-1, h)
    out = _moe(x, gate_w, Wg, Wu, Wd)
    return out.reshape(b, s, h)


# speedup vs baseline: 1.3108x; 1.0221x over previous
import functools

import jax
import jax.numpy as jnp
from jax.experimental import pallas as pl
from jax.experimental.pallas import tpu as pltpu
from jax.experimental.pallas import tpu_sc as plsc

HIDDEN = 1024
NUM_EXPERTS = 8
TOP_K = 2
MOE_FF = 512

RBT = 512
BT = 256
BLK = 256
T_TOK = 2048
A = T_TOK * TOP_K
NB_MAX = A // BLK + NUM_EXPERTS
NP_MAX = NB_MAX * BLK

_NC, _NS = 2, 16
_NW = _NC * _NS
_CH = 64


def _pack_halves16(a16):
    n = a16.shape[1] // 2
    lo = jax.lax.bitcast_convert_type(a16[:, :n], jnp.uint16).astype(jnp.uint32)
    hi = jax.lax.bitcast_convert_type(a16[:, n:], jnp.uint16).astype(jnp.uint32)
    return jax.lax.bitcast_convert_type((hi << 16) | lo, jnp.int32)


def _router_kernel(x_ref, gw_ref, w_ref, pos_ref, counts_ref, xpk_ref, run_ref):
    t = pl.program_id(0)

    @pl.when(t == 0)
    def _():
        run_ref[...] = jnp.zeros((1, NUM_EXPERTS), jnp.float32)

    xb = x_ref[...]
    logits = jnp.dot(xb, gw_ref[...].T, preferred_element_type=jnp.float32)
    m = jnp.max(logits, axis=1, keepdims=True)
    p = jnp.exp(logits - m)
    prob = p / jnp.sum(p, axis=1, keepdims=True)
    iota_e = jax.lax.broadcasted_iota(jnp.int32, prob.shape, 1)
    i1 = jnp.argmax(prob, axis=1).astype(jnp.int32)
    w1 = jnp.max(prob, axis=1)
    masked = jnp.where(iota_e == i1[:, None], -1.0, prob)
    i2 = jnp.argmax(masked, axis=1).astype(jnp.int32)
    w2 = jnp.max(masked, axis=1)
    s = w1 + w2
    w_ref[...] = jnp.stack([w1 / s, w2 / s], axis=1)
    xpk_ref[...] = _pack_halves16(xb.astype(jnp.bfloat16))

    oh1 = (iota_e == i1[:, None]).astype(jnp.float32)
    oh2 = (iota_e == i2[:, None]).astype(jnp.float32)
    cnt = oh1 + oh2
    r_i = jax.lax.broadcasted_iota(jnp.int32, (RBT, RBT), 0)
    c_i = jax.lax.broadcasted_iota(jnp.int32, (RBT, RBT), 1)
    tril = (c_i < r_i).astype(jnp.float32)
    excl = jnp.dot(tril, cnt, preferred_element_type=jnp.float32) + run_ref[...]
    rank1 = jnp.sum(oh1 * excl, axis=1)
    rank2 = jnp.sum(oh2 * excl, axis=1)
    pos1 = i1 * T_TOK + rank1.astype(jnp.int32)
    pos2 = i2 * T_TOK + rank2.astype(jnp.int32)
    pos_ref[...] = jnp.stack([pos1, pos2], axis=1)
    run_new = run_ref[...] + jnp.sum(cnt, axis=0, keepdims=True)
    run_ref[...] = run_new
    counts_ref[...] = run_new.astype(jnp.int32)


def _router(x, gate_w):
    T, H = x.shape
    E = NUM_EXPERTS
    return pl.pallas_call(
        _router_kernel,
        grid=(T // RBT,),
        in_specs=[
            pl.BlockSpec((RBT, H), lambda t: (t, 0)),
            pl.BlockSpec((E, H), lambda t: (0, 0)),
        ],
        out_specs=[
            pl.BlockSpec((RBT, TOP_K), lambda t: (t, 0)),
            pl.BlockSpec((RBT, TOP_K), lambda t: (t, 0)),
            pl.BlockSpec((1, E), lambda t: (0, 0)),
            pl.BlockSpec((RBT, H // 2), lambda t: (t, 0)),
        ],
        out_shape=[
            jax.ShapeDtypeStruct((T, TOP_K), jnp.float32),
            jax.ShapeDtypeStruct((T, TOP_K), jnp.int32),
            jax.ShapeDtypeStruct((1, E), jnp.int32),
            jax.ShapeDtypeStruct((T, H // 2), jnp.int32),
        ],
        scratch_shapes=[pltpu.VMEM((1, E), jnp.float32)],
    )(x, gate_w)


def _plan(counts):
    c = counts[0]
    nact = (c + BLK - 1) // BLK
    cend = jnp.cumsum(nact)
    cstart = cend - nact
    nblocks = cend[-1].astype(jnp.int32)
    s = jnp.arange(NB_MAX, dtype=jnp.int32)
    sc = jnp.minimum(s, nblocks - 1)
    block_e = jnp.sum(sc[:, None] >= cend[None, :], axis=1).astype(jnp.int32)
    bidx = (block_e * (T_TOK // BLK) + sc - cstart[block_e]).astype(jnp.int32)
    return nblocks, block_e, bidx


def _sc_dispatch(x, pos0, pos1):
    T, H = x.shape
    bpw = T // _NW
    mesh = plsc.VectorSubcoreMesh(core_axis_name="c", subcore_axis_name="s")

    @functools.partial(
        pl.kernel, mesh=mesh,
        out_type=jax.ShapeDtypeStruct((NUM_EXPERTS * T_TOK, H), x.dtype),

        scratch_types=[
            pltpu.VMEM((bpw,), jnp.int32),
            pltpu.VMEM((bpw,), jnp.int32),
            pltpu.VMEM((bpw, H), x.dtype),
            pltpu.SemaphoreType.DMA,
        ],
    )
    def k(x_hbm, p0_hbm, p1_hbm, xs_hbm, i0_v, i1_v, rows_v, sem):
        wid = jax.lax.axis_index("s") * _NC + jax.lax.axis_index("c")
        base = wid * bpw
        pltpu.sync_copy(p0_hbm.at[pl.ds(base, bpw)], i0_v)
        pltpu.sync_copy(p1_hbm.at[pl.ds(base, bpw)], i1_v)
        pltpu.sync_copy(x_hbm.at[pl.ds(base, bpw)], rows_v)
        pltpu.async_copy(rows_v, xs_hbm.at[i0_v], sem).wait()
        pltpu.async_copy(rows_v, xs_hbm.at[i1_v], sem).wait()

    return k(x, pos0, pos1)


def _sc_gather(table, idx):
    B = idx.shape[0]
    V, D = table.shape
    bpw = B // _NW
    nch = bpw // _CH
    mesh = plsc.VectorSubcoreMesh(core_axis_name="c", subcore_axis_name="s")

    @functools.partial(
        pl.kernel, mesh=mesh,
        out_type=jax.ShapeDtypeStruct((B, D), table.dtype),
        scratch_types=[
            pltpu.VMEM((_CH,), jnp.int32),
            pltpu.VMEM((_CH, D), table.dtype),
            pltpu.SemaphoreType.DMA,
        ],
    )
    def k(table_hbm, idx_hbm, out_hbm, idx_v, rows_v, sem):
        wid = jax.lax.axis_index("s") * _NC + jax.lax.axis_index("c")
        base = wid * bpw
        for c in range(nch):
            off = base + c * _CH
            pltpu.sync_copy(idx_hbm.at[pl.ds(off, _CH)], idx_v)
            pltpu.async_copy(table_hbm.at[idx_v], rows_v, sem).wait()
            pltpu.sync_copy(rows_v, out_hbm.at[pl.ds(off, _CH)])

    return k(table, idx)


def _gm_kernel(be_ref, bi_ref, nb_ref, xs_ref, wg_ref, wu_ref, wd_ref, y_ref):
    i = pl.program_id(0)

    @pl.when(i < nb_ref[0])
    def _():
        v = jax.lax.bitcast_convert_type(xs_ref[...], jnp.uint32)
        xlo = jax.lax.bitcast_convert_type(
            (v & 0xFFFF).astype(jnp.uint16), jnp.bfloat16)
        xhi = jax.lax.bitcast_convert_type(
            (v >> 16).astype(jnp.uint16), jnp.bfloat16)
        xb = jnp.concatenate([xlo, xhi], axis=1)
        g = jnp.dot(xb, wg_ref[0].astype(jnp.bfloat16).T,
                    preferred_element_type=jnp.float32)
        u = jnp.dot(xb, wu_ref[0].astype(jnp.bfloat16).T,
                    preferred_element_type=jnp.float32)
        act = g * jax.nn.sigmoid(g) * u
        o = jnp.dot(act.astype(jnp.bfloat16), wd_ref[0].astype(jnp.bfloat16).T,
                    preferred_element_type=jnp.float32)
        o16 = o.astype(jnp.bfloat16)
        lo = jax.lax.bitcast_convert_type(
            o16[:, :HIDDEN // 2], jnp.uint16).astype(jnp.uint32)
        hi = jax.lax.bitcast_convert_type(
            o16[:, HIDDEN // 2:], jnp.uint16).astype(jnp.uint32)
        y_ref[...] = jax.lax.bitcast_convert_type((hi << 16) | lo, jnp.int32)


def _grouped_mlp(xs, Wg, Wu, Wd, nblocks, block_e, bidx):
    H, F = HIDDEN, MOE_FF
    grid_spec = pltpu.PrefetchScalarGridSpec(
        num_scalar_prefetch=3,
        grid=(NB_MAX,),
        in_specs=[
            pl.BlockSpec((BLK, H // 2), lambda i, be, bi, nb: (bi[i], 0)),
            pl.BlockSpec((1, F, H), lambda i, be, bi, nb: (be[i], 0, 0)),
            pl.BlockSpec((1, F, H), lambda i, be, bi, nb: (be[i], 0, 0)),
            pl.BlockSpec((1, H, F), lambda i, be, bi, nb: (be[i], 0, 0)),
        ],
        out_specs=pl.BlockSpec((BLK, H // 2), lambda i, be, bi, nb: (bi[i], 0)),
    )
    return pl.pallas_call(
        _gm_kernel,
        grid_spec=grid_spec,
        out_shape=jax.ShapeDtypeStruct((NUM_EXPERTS * T_TOK, H // 2), jnp.int32),
    )(block_e, bidx, nblocks.reshape(1), xs, Wg, Wu, Wd)


def _unpack_halves(v_i32):
    v = jax.lax.bitcast_convert_type(v_i32, jnp.uint32)
    lo = jax.lax.bitcast_convert_type(
        (v & 0xFFFF).astype(jnp.uint16), jnp.bfloat16).astype(jnp.float32)
    hi = jax.lax.bitcast_convert_type(
        (v >> 16).astype(jnp.uint16), jnp.bfloat16).astype(jnp.float32)
    return lo, hi


def _combine_kernel(w_ref, y0_ref, y1_ref, out_ref):
    lo0, hi0 = _unpack_halves(y0_ref[...])
    lo1, hi1 = _unpack_halves(y1_ref[...])
    w0 = w_ref[:, 0:1]
    w1 = w_ref[:, 1:2]
    out_ref[:, :HIDDEN // 2] = w0 * lo0 + w1 * lo1
    out_ref[:, HIDDEN // 2:] = w0 * hi0 + w1 * hi1


def _combine(w, yy, T):
    Hw = yy.shape[1]
    nt = T // BT
    return pl.pallas_call(
        _combine_kernel,
        grid=(nt,),
        in_specs=[
            pl.BlockSpec((BT, TOP_K), lambda t: (t, 0)),
            pl.BlockSpec((BT, Hw), lambda t: (t, 0)),
            pl.BlockSpec((BT, Hw), lambda t, _nt=nt: (t + _nt, 0)),
        ],
        out_specs=pl.BlockSpec((BT, HIDDEN), lambda t: (t, 0)),
        out_shape=jax.ShapeDtypeStruct((T, HIDDEN), jnp.float32),
    )(w, yy, yy)


@jax.jit
def _moe(x, gate_w, Wg, Wu, Wd):
    T = x.shape[0]
    w, pos, counts, xpk = _router(x, gate_w)
    nblocks, block_e, bidx = _plan(counts)
    xs = _sc_dispatch(xpk, pos[:, 0], pos[:, 1])
    y = _grouped_mlp(xs, Wg, Wu, Wd, nblocks, block_e, bidx)
    yy = _sc_gather(y, pos.T.reshape(-1))
    return _combine(w, yy, T)


def kernel(hidden_states, gate_w, Wg, Wu, Wd):
    b, s, h = hidden_states.shape
    x = hidden_states.reshape(-1, h)
    out = _moe(x, gate_w, Wg, Wu, Wd)
    return out.reshape(b, s, h)
